# Initial kernel scaffold; baseline (speedup 1.0000x reference)
import functools

import jax
import jax.numpy as jnp
from jax import lax
from jax.experimental import pallas as pl
from jax.experimental.pallas import tpu as pltpu
from jax.experimental.pallas import tpu_sc as plsc

B = 16384
D = 64
NEG = 20

NUM_CORES = 2
NUM_SUBCORES = 16
NW = NUM_CORES * NUM_SUBCORES
LANES = 16

CB = 64
CHUNKS = B // (NW * CB)
GROUPS = CB // LANES
NEG_SLICES = CB * NEG // 128


def _sc_body(c_hbm, o_hbm, n_hbm, win_hbm, wout_hbm, pos_out, neg_out,
             c_idx_v, o_idx_v, n_idx_v, c_rows, o_rows, n_rows,
             pos_buf, neg_buf, sem):
  wid = lax.axis_index("s") * NUM_CORES + lax.axis_index("c")
  for chunk in range(CHUNKS):
    base = wid * (CHUNKS * CB) + chunk * CB
    pltpu.sync_copy(c_hbm.at[pl.ds(base, CB)], c_idx_v)
    pltpu.sync_copy(o_hbm.at[pl.ds(base, CB)], o_idx_v)
    pltpu.sync_copy(n_hbm.at[pl.ds(wid * CHUNKS * NEG_SLICES
                                   + chunk * NEG_SLICES, NEG_SLICES)],
                    n_idx_v)
    cps = [pltpu.async_copy(win_hbm.at[c_idx_v], c_rows, sem),
           pltpu.async_copy(wout_hbm.at[o_idx_v], o_rows, sem)]
    for j in range(NEG_SLICES):
      cps.append(pltpu.async_copy(wout_hbm.at[n_idx_v.at[j]],
                                  n_rows.at[pl.ds(j * 128, 128)], sem))
    for cp in cps:
      cp.wait()

    for g in range(GROUPS):
      row16 = lax.iota(jnp.int32, 16) + g * LANES
      row20 = row16 * NEG

      def dbody(d, accs, row16=row16, row20=row20):
        d16 = jnp.full((LANES,), d, jnp.int32)
        cv = plsc.load_gather(c_rows, [row16, d16])
        ov = plsc.load_gather(o_rows, [row16, d16])
        news = [accs[0] + cv * ov]
        for k in range(NEG):
          nv = plsc.load_gather(n_rows, [row20 + k, d16])
          news.append(accs[k + 1] + nv * cv)
        return tuple(news)

      init = tuple(jnp.zeros((LANES,), jnp.float32) for _ in range(NEG + 1))
      accs = lax.fori_loop(0, D, dbody, init)
      pos_buf[pl.ds(g * LANES, LANES)] = accs[0]
      for k in range(NEG):
        plsc.store_scatter(neg_buf, [row16, jnp.full((LANES,), k, jnp.int32)],
                           accs[k + 1])

    pltpu.sync_copy(pos_buf, pos_out.at[pl.ds(base, CB)])
    pltpu.sync_copy(neg_buf, neg_out.at[pl.ds(base, CB)])


_sc_scores = functools.partial(
    pl.kernel,
    out_type=[jax.ShapeDtypeStruct((B,), jnp.float32),
              jax.ShapeDtypeStruct((B, NEG), jnp.float32)],
    mesh=plsc.VectorSubcoreMesh(core_axis_name="c", subcore_axis_name="s"),
    scratch_types=[
        pltpu.VMEM((CB,), jnp.int32),
        pltpu.VMEM((CB,), jnp.int32),
        pltpu.VMEM((NEG_SLICES, 128), jnp.int32),
        pltpu.VMEM((CB, D), jnp.float32),
        pltpu.VMEM((CB, D), jnp.float32),
        pltpu.VMEM((CB * NEG, D), jnp.float32),
        pltpu.VMEM((CB,), jnp.float32),
        pltpu.VMEM((CB, NEG), jnp.float32),
        pltpu.SemaphoreType.DMA,
    ],
)(_sc_body)


def _loss_body(pos_ref, neg_ref, out_ref):
  def logsig(x):
    return -(jnp.maximum(-x, 0.0) + jnp.log1p(jnp.exp(-jnp.abs(x))))

  pos = pos_ref[...]
  neg = neg_ref[...]
  out_ref[0, 0] = -(jnp.mean(logsig(pos)) + jnp.mean(logsig(-neg)))


_loss_tc = pl.pallas_call(
    _loss_body,
    out_shape=jax.ShapeDtypeStruct((1, 1), jnp.float32),
)


def kernel(center_words, outside_words, negative_samples, W_in, W_out):
  c = center_words.astype(jnp.int32)
  o = outside_words.astype(jnp.int32)
  n = negative_samples.astype(jnp.int32).reshape(B * NEG // 128, 128)
  pos, neg = _sc_scores(c, o, n, W_in, W_out)
  loss = _loss_tc(pos.reshape(128, B // 128), neg.reshape(B * NEG // 128, 128))
  return loss[0, 0]


# speedup vs baseline: 4.1521x; 4.1521x over previous
import functools

import jax
import jax.numpy as jnp
from jax import lax
from jax.experimental import pallas as pl
from jax.experimental.pallas import tpu as pltpu
from jax.experimental.pallas import tpu_sc as plsc

B = 16384
D = 64
NEG = 20
NDOT = NEG + 1
VOCAB = 1000000

NUM_CORES = 2
NUM_SUBCORES = 16
NW = NUM_CORES * NUM_SUBCORES
LANES = 16

CB = 32
CHUNKS = B // (NW * CB)
ONC = CB * NDOT
SCOL = 32


def _sc_body(gc_hbm, gon_hbm, oc_hbm, oon_hbm, win_hbm, wout_hbm, s_out,
             c_gidx, on_gidx, c_off, on_off, smc, smon,
             c_rows, on_rows, scores, red, srow, sem):
  wid = lax.axis_index("s") * NUM_CORES + lax.axis_index("c")

  for r in range(4):
    red[r, pl.ds(16, 16)] = jnp.zeros((16,), jnp.float32)

  def chunk_body(chunk, _):
    base = wid * (CHUNKS * CB) + chunk * CB
    pltpu.sync_copy(gc_hbm.at[pl.ds(base, CB)], c_gidx)
    pltpu.sync_copy(gon_hbm.at[pl.ds(base * NDOT, ONC)], on_gidx)
    pltpu.sync_copy(oc_hbm.at[pl.ds(base, CB)], c_off)
    pltpu.sync_copy(oon_hbm.at[pl.ds(base * NDOT, ONC)], on_off)

    cps = [pltpu.async_copy(win_hbm.at[c_gidx], c_rows, sem)]
    for j in range(ONC // 128):
      cps.append(pltpu.async_copy(wout_hbm.at[on_gidx.at[pl.ds(j * 128, 128)]],
                                  on_rows.at[pl.ds(j * 128, 128)], sem))
    rem = ONC % 128
    if rem:
      cps.append(pltpu.async_copy(
          wout_hbm.at[on_gidx.at[pl.ds(ONC - rem, rem)]],
          on_rows.at[pl.ds(ONC - rem, rem)], sem))

    for g in range(CB // 16):
      v = c_off[pl.ds(g * 16, 16)]
      for j in range(16):
        smc[g * 16 + j] = v[j]
    for g in range(ONC // 16):
      v = on_off[pl.ds(g * 16, 16)]
      for j in range(16):
        smon[g * 16 + j] = v[j]

    for cp in cps:
      cp.wait()

    def elem_body(i, _):
      offc = smc[i]
      c0 = c_rows[i, pl.ds(offc, 16)]
      c1 = c_rows[i, pl.ds(offc + 16, 16)]
      c2 = c_rows[i, pl.ds(offc + 32, 16)]
      c3 = c_rows[i, pl.ds(offc + 48, 16)]
      rbase = i * NDOT
      for t in range(NDOT):
        offx = smon[rbase + t]
        r = rbase + t
        q = (c0 * on_rows[r, pl.ds(offx, 16)]
             + c1 * on_rows[r, pl.ds(offx + 16, 16)]
             + c2 * on_rows[r, pl.ds(offx + 32, 16)]
             + c3 * on_rows[r, pl.ds(offx + 48, 16)])
        rb = red.at[t % 4]
        rb[pl.ds(0, 16)] = q
        q = q + rb[pl.ds(8, 16)]
        rb[pl.ds(0, 16)] = q
        q = q + rb[pl.ds(4, 16)]
        rb[pl.ds(0, 16)] = q
        q = q + rb[pl.ds(2, 16)]
        rb[pl.ds(0, 16)] = q
        q = q + rb[pl.ds(1, 16)]
        srow[pl.ds(t, 16)] = q
      scores[i, pl.ds(0, 16)] = srow[pl.ds(0, 16)]
      scores[i, pl.ds(16, 16)] = srow[pl.ds(16, 16)]
      return 0

    lax.fori_loop(0, CB, elem_body, 0)
    pltpu.sync_copy(scores, s_out.at[pl.ds(base, CB)])
    return 0

  lax.fori_loop(0, CHUNKS, chunk_body, 0)


_sc_scores = functools.partial(
    pl.kernel,
    out_type=jax.ShapeDtypeStruct((B, SCOL), jnp.float32),
    mesh=plsc.VectorSubcoreMesh(core_axis_name="c", subcore_axis_name="s"),
    scratch_types=[
        pltpu.VMEM((CB,), jnp.int32),
        pltpu.VMEM((ONC,), jnp.int32),
        pltpu.VMEM((CB,), jnp.int32),
        pltpu.VMEM((ONC,), jnp.int32),
        pltpu.SMEM((CB,), jnp.int32),
        pltpu.SMEM((ONC,), jnp.int32),
        pltpu.VMEM((CB, 128), jnp.float32),
        pltpu.VMEM((ONC, 128), jnp.float32),
        pltpu.VMEM((CB, SCOL), jnp.float32),
        pltpu.VMEM((4, 32), jnp.float32),
        pltpu.VMEM((48,), jnp.float32),
        pltpu.SemaphoreType.DMA,
    ],
)(_sc_body)


def _loss_body(s_ref, out_ref):
  def logsig(x):
    return -(jnp.maximum(-x, 0.0) + jnp.log1p(jnp.exp(-jnp.abs(x))))

  x = s_ref[...]
  col = lax.broadcasted_iota(jnp.int32, x.shape, 1) % SCOL
  pos_sum = jnp.sum(jnp.where(col == 0, logsig(x), 0.0))
  neg_sum = jnp.sum(jnp.where((col >= 1) & (col <= NEG), logsig(-x), 0.0))
  loss = -(pos_sum / B + neg_sum / (B * NEG))
  out_ref[...] = jnp.full((1, 1), loss, jnp.float32)


_loss_tc = pl.pallas_call(
    _loss_body,
    out_shape=jax.ShapeDtypeStruct((1, 1), jnp.float32),
)


def kernel(center_words, outside_words, negative_samples, W_in, W_out):
  c = center_words.astype(jnp.int32)
  on = jnp.concatenate(
      [outside_words.astype(jnp.int32)[:, None],
       negative_samples.astype(jnp.int32)], axis=1).reshape(B * NDOT)
  gc = c >> 1
  oc = (c & 1) * D
  gon = on >> 1
  oon = (on & 1) * D
  win2 = W_in.reshape(VOCAB // 2, 2 * D)
  wout2 = W_out.reshape(VOCAB // 2, 2 * D)
  scores = _sc_scores(gc, gon, oc, oon, win2, wout2)
  loss = _loss_tc(scores.reshape(B * SCOL // 128, 128))
  return loss[0, 0]
